# baseline (speedup 1.0000x reference)
import jax
import jax.numpy as jnp
from jax.experimental import pallas as pl
from jax.experimental.pallas import tpu as pltpu

_NC = 7680


def _cwr_head_kernel(x_ref, wt_ref, b_ref, o_ref):
    y = jax.lax.dot_general(
        x_ref[:], wt_ref[:],
        dimension_numbers=(((1,), (0,)), ((), ())),
        preferred_element_type=jnp.float32,
    )
    o_ref[:] = y + b_ref[:]


def kernel(x, W, b):
    batch, k = x.shape
    n = W.shape[0]
    return pl.pallas_call(
        _cwr_head_kernel,
        grid=(pl.cdiv(n, _NC),),
        in_specs=[
            pl.BlockSpec((batch, k), lambda i: (0, 0)),
            pl.BlockSpec((k, _NC), lambda i: (0, i)),
            pl.BlockSpec((1, _NC), lambda i: (0, i)),
        ],
        out_specs=pl.BlockSpec((batch, _NC), lambda i: (0, i)),
        out_shape=jax.ShapeDtypeStruct((batch, n), jnp.float32),
        compiler_params=pltpu.CompilerParams(
            dimension_semantics=("arbitrary",),
            vmem_limit_bytes=100 * 1024 * 1024,
        ),
    )(x, W.T, b.reshape(1, n))
